# baseline (speedup 1.0000x reference)
import functools
import math

import jax
import jax.numpy as jnp
from jax.experimental import pallas as pl
from jax.experimental.pallas import tpu as pltpu


def _rowsum_kernel(x_ref, o_ref):
    o_ref[0] = jnp.sum(x_ref[0], axis=-1)


def _softmax_last(z):
    m = jnp.max(z, axis=-1, keepdims=True)
    e = jnp.exp(z - m)
    return e / jnp.sum(e, axis=-1, keepdims=True)


def _row_sum_treed(e):
    n = e.shape[-1]
    acc = e[:, 0:128]
    for i in range(1, n // 128):
        acc = acc + e[:, i * 128:(i + 1) * 128]
    t = acc[:, 0:8]
    for i in range(1, 16):
        t = t + acc[:, i * 8:(i + 1) * 8]
    u = t[:, 0:4] + t[:, 4:8]
    v = u[:, 0:2] + u[:, 2:4]
    return v[:, 0:1] + v[:, 1:2]


def _bf(v):
    return v.astype(jnp.bfloat16).astype(jnp.float32)


def _graph_kernel(params_ref, xs_tile_ref, xs_full_ref, mem_ref, o_ref,
                  *, scale, k):
    w0 = params_ref[0]
    w1 = params_ref[1]
    bias = params_ref[2]

    lhs = xs_tile_ref[0]
    rhs_m = mem_ref[...]
    rhs_x = xs_full_ref[0]

    dn = (((0,), (0,)), ((), ()))
    z1 = jax.lax.dot_general(lhs, rhs_m, dn,
                             preferred_element_type=jnp.float32) / scale
    z2 = jax.lax.dot_general(lhs, rhs_x, dn,
                             preferred_element_type=jnp.float32) / scale

    a1 = _softmax_last(jnp.maximum(z1, 0.0))
    a2 = _softmax_last(jnp.maximum(z2, 0.0))

    lin = (_bf(a1) * _bf(w0) + _bf(a2) * _bf(w1)) + bias

    m3 = jnp.max(lin, axis=-1, keepdims=True)
    e3 = jnp.exp(lin - m3)
    z3 = _row_sum_treed(e3)
    adj = e3 / z3

    tn = adj.shape[0]
    n = adj.shape[1]

    p = jnp.full((tn, 1), 0x38000000, jnp.int32)
    for bit in range(25, -1, -1):
        q = p | jnp.int32(1 << bit)
        t = jax.lax.bitcast_convert_type(q, jnp.float32)
        cnt = jnp.sum((adj >= t).astype(jnp.int32), axis=-1, keepdims=True)
        p = jnp.where(cnt >= k, q, p)
    thresh = jax.lax.bitcast_convert_type(p, jnp.float32)

    gt = adj > thresh
    eq = adj == thresh
    cnt_gt = jnp.sum(gt.astype(jnp.int32), axis=-1, keepdims=True)
    needed = k - cnt_gt

    iota = jax.lax.broadcasted_iota(jnp.int32, (tn, n), 1)
    c = jnp.zeros((tn, 1), jnp.int32)
    for bit in range(11, -1, -1):
        q = c | jnp.int32(1 << bit)
        cnt = jnp.sum((eq & (iota < q)).astype(jnp.int32),
                      axis=-1, keepdims=True)
        c = jnp.where(cnt <= needed, q, c)

    keep = gt | (eq & (iota < c))
    o_ref[0] = jnp.where(keep, adj, 0.0)


@jax.jit
def kernel(x, memory, fc_w, fc_b):
    b, c, n, t = x.shape
    scale = math.sqrt(c)
    k = int(n * 0.8)

    tn_sum = 256
    xs = pl.pallas_call(
        _rowsum_kernel,
        grid=(b, n // tn_sum),
        in_specs=[pl.BlockSpec((1, c, tn_sum, t), lambda bi, i: (bi, 0, i, 0))],
        out_specs=pl.BlockSpec((1, c, tn_sum), lambda bi, i: (bi, 0, i)),
        out_shape=jax.ShapeDtypeStruct((b, c, n), jnp.float32),
    )(x)

    params = jnp.concatenate([fc_w.reshape(-1), fc_b.reshape(-1)])

    tn = 512
    out = pl.pallas_call(
        functools.partial(_graph_kernel, scale=scale, k=k),
        grid=(b, n // tn),
        in_specs=[
            pl.BlockSpec(memory_space=pltpu.SMEM),
            pl.BlockSpec((1, c, tn), lambda bi, i: (bi, 0, i)),
            pl.BlockSpec((1, c, n), lambda bi, i: (bi, 0, 0)),
            pl.BlockSpec((c, n), lambda bi, i: (0, 0)),
        ],
        out_specs=pl.BlockSpec((1, tn, n), lambda bi, i: (bi, i, 0)),
        out_shape=jax.ShapeDtypeStruct((b, n, n), jnp.float32),
    )(params, xs, xs, memory)
    return out


# speedup vs baseline: 1.0280x; 1.0280x over previous
import functools
import math

import jax
import jax.numpy as jnp
from jax.experimental import pallas as pl
from jax.experimental.pallas import tpu as pltpu


def _rowsum_kernel(x_ref, o_ref):
    o_ref[0] = jnp.sum(x_ref[0], axis=-1)


def _softmax_last(z):
    m = jnp.max(z, axis=-1, keepdims=True)
    e = jnp.exp(z - m)
    return e / jnp.sum(e, axis=-1, keepdims=True)


def _row_sum_treed(e):
    n = e.shape[-1]
    acc = e[:, 0:128]
    for i in range(1, n // 128):
        acc = acc + e[:, i * 128:(i + 1) * 128]
    t = acc[:, 0:8]
    for i in range(1, 16):
        t = t + acc[:, i * 8:(i + 1) * 8]
    u = t[:, 0:4] + t[:, 4:8]
    v = u[:, 0:2] + u[:, 2:4]
    return v[:, 0:1] + v[:, 1:2]


def _bf(v):
    return v.astype(jnp.bfloat16).astype(jnp.float32)


def _graph_kernel(params_ref, xs_tile_ref, xs_full_ref, mem_ref, o_ref,
                  *, scale, k):
    w0 = params_ref[0]
    w1 = params_ref[1]
    bias = params_ref[2]

    lhs = xs_tile_ref[0]
    rhs_m = mem_ref[...]
    rhs_x = xs_full_ref[0]

    dn = (((0,), (0,)), ((), ()))
    z1 = jax.lax.dot_general(lhs, rhs_m, dn,
                             preferred_element_type=jnp.float32) / scale
    z2 = jax.lax.dot_general(lhs, rhs_x, dn,
                             preferred_element_type=jnp.float32) / scale

    a1 = _softmax_last(jnp.maximum(z1, 0.0))
    a2 = _softmax_last(jnp.maximum(z2, 0.0))

    lin = (_bf(a1) * _bf(w0) + _bf(a2) * _bf(w1)) + bias

    m3 = jnp.max(lin, axis=-1, keepdims=True)
    e3 = jnp.exp(lin - m3)
    z3 = _row_sum_treed(e3)
    adj = e3 / z3

    tn = adj.shape[0]
    n = adj.shape[1]

    p = jnp.full((tn, 1), 0x38000000, jnp.int32)
    for bit in range(25, -1, -1):
        q = p | jnp.int32(1 << bit)
        t = jax.lax.bitcast_convert_type(q, jnp.float32)
        cnt = jnp.sum((adj >= t).astype(jnp.int32), axis=-1, keepdims=True)
        p = jnp.where(cnt >= k, q, p)
    thresh = jax.lax.bitcast_convert_type(p, jnp.float32)

    gt = adj > thresh
    eq = adj == thresh
    cnt_gt = jnp.sum(gt.astype(jnp.int32), axis=-1, keepdims=True)
    needed = k - cnt_gt

    s = eq.astype(jnp.int32)
    incl = s
    d = 1
    while d < n:
        shifted = jnp.concatenate(
            [jnp.zeros((tn, d), jnp.int32), incl[:, :n - d]], axis=1)
        incl = incl + shifted
        d *= 2
    excl = incl - s

    keep = gt | (eq & (excl < needed))
    o_ref[0] = jnp.where(keep, adj, 0.0)


@jax.jit
def kernel(x, memory, fc_w, fc_b):
    b, c, n, t = x.shape
    scale = math.sqrt(c)
    k = int(n * 0.8)

    tn_sum = 512
    xs = pl.pallas_call(
        _rowsum_kernel,
        grid=(b, n // tn_sum),
        in_specs=[pl.BlockSpec((1, c, tn_sum, t), lambda bi, i: (bi, 0, i, 0))],
        out_specs=pl.BlockSpec((1, c, tn_sum), lambda bi, i: (bi, 0, i)),
        out_shape=jax.ShapeDtypeStruct((b, c, n), jnp.float32),
    )(x)

    params = jnp.concatenate([fc_w.reshape(-1), fc_b.reshape(-1)])

    tn = 512
    out = pl.pallas_call(
        functools.partial(_graph_kernel, scale=scale, k=k),
        grid=(b, n // tn),
        in_specs=[
            pl.BlockSpec(memory_space=pltpu.SMEM),
            pl.BlockSpec((1, c, tn), lambda bi, i: (bi, 0, i)),
            pl.BlockSpec((1, c, n), lambda bi, i: (bi, 0, 0)),
            pl.BlockSpec((c, n), lambda bi, i: (0, 0)),
        ],
        out_specs=pl.BlockSpec((1, tn, n), lambda bi, i: (bi, i, 0)),
        out_shape=jax.ShapeDtypeStruct((b, n, n), jnp.float32),
    )(params, xs, xs, memory)
    return out
